# baseline (speedup 1.0000x reference)
import functools

import jax
import jax.numpy as jnp
from jax import lax
from jax.experimental import pallas as pl
from jax.experimental.pallas import tpu as pltpu
from jax.experimental.pallas import tpu_sc as plsc

B = 4096
LG = 200
LS = 200
D = 128
SV = 1000
TC_ = 100
NW = 32
CH = 128
NBUF = 4
N_LOOK = B * TC_
PER_W = N_LOOK // NW
NCH = PER_W // CH

BM = 256
TT = 20
NT = TC_ // TT

BPW = B // NW


def _sc_gather(idx, tab):
    mesh = plsc.VectorSubcoreMesh(core_axis_name="c", subcore_axis_name="s")

    @functools.partial(
        pl.kernel,
        out_type=jax.ShapeDtypeStruct((N_LOOK, D), jnp.float32),
        mesh=mesh,
        scratch_types=[
            pltpu.VMEM((NCH, CH), jnp.int32),
        ] + [pltpu.VMEM((CH, D), jnp.float32)] * NBUF
          + [pltpu.SemaphoreType.DMA] * (2 * NBUF),
    )
    def k(idx_h, tab_h, out_h, idx_v, *bufsem):
        bufs = bufsem[:NBUF]
        gsems = bufsem[NBUF:2 * NBUF]
        wsems = bufsem[2 * NBUF:]
        wid = lax.axis_index("s") * 2 + lax.axis_index("c")
        base = wid * PER_W

        pltpu.sync_copy(idx_h.at[wid], idx_v)

        def fire(s, b):
            pltpu.async_copy(tab_h.at[idx_v.at[s]], bufs[b], gsems[b])

        def drain(s, b):
            pltpu.make_async_copy(
                tab_h.at[idx_v.at[s]], bufs[b], gsems[b]).wait()

        def wb(s, b):
            pltpu.async_copy(
                bufs[b], out_h.at[pl.ds(base + s * CH, CH)], wsems[b])

        def wb_wait(s, b):
            pltpu.make_async_copy(
                bufs[b], out_h.at[pl.ds(base + s * CH, CH)], wsems[b]).wait()

        for j in range(NBUF - 1):
            fire(j, j)
        for s in range(NCH):
            b = s % NBUF
            drain(s, b)
            wb(s, b)
            nxt = s + NBUF - 1
            if nxt < NCH:
                nb = nxt % NBUF
                if s >= 1:
                    wb_wait(s - 1, nb)
                fire(nxt, nb)
        for s in range(NCH - NBUF, NCH):
            wb_wait(s, s % NBUF)

    return k(idx, tab)


def _m_prep(stab, W1s3):
    def body(x_r, w_r, out_r):
        w = jnp.concatenate([w_r[0], w_r[1]], axis=-1)
        out_r[0] = jnp.dot(x_r[...], w, preferred_element_type=jnp.float32)

    return pl.pallas_call(
        body,
        grid=(LS // 2,),
        in_specs=[
            pl.BlockSpec((SV, D), lambda g: (0, 0)),
            pl.BlockSpec((2, D, 64), lambda g: (g, 0, 0)),
        ],
        out_specs=pl.BlockSpec((1, SV, D), lambda g: (g, 0, 0)),
        out_shape=jax.ShapeDtypeStruct((LS // 2, SV, D), jnp.float32),
    )(stab, W1s3)


def _sc_msum(idxb, mtab):
    mesh = plsc.VectorSubcoreMesh(core_axis_name="c", subcore_axis_name="s")

    @functools.partial(
        pl.kernel,
        out_type=jax.ShapeDtypeStruct((B, 64), jnp.float32),
        mesh=mesh,
        scratch_types=[
            pltpu.VMEM((BPW, LS), jnp.int32),
        ] + [pltpu.VMEM((CH, D), jnp.float32)] * 4
          + [pltpu.VMEM((1, 64), jnp.float32)] * 2
          + [pltpu.SemaphoreType.DMA] * 6,
    )
    def k(idx_h, m_h, out_h, idx_v, buf0, buf1, buf2, buf3,
          hA, hB, gsem0, gsem1, gsem2, gsem3, wsemA, wsemB):
        wid = lax.axis_index("s") * 2 + lax.axis_index("c")
        base = wid * BPW

        pltpu.sync_copy(idx_h.at[wid], idx_v)

        bufs = (buf0, buf1, buf2, buf3)
        gsems = (gsem0, gsem1, gsem2, gsem3)

        def fire(bb, half, par):
            if half == 0:
                pltpu.async_copy(m_h.at[idx_v.at[bb, pl.ds(0, 128)]],
                                 bufs[par].at[pl.ds(0, 128)], gsems[par])
            else:
                pltpu.async_copy(m_h.at[idx_v.at[bb, pl.ds(128, LS - 128)]],
                                 bufs[par].at[pl.ds(0, LS - 128)], gsems[par])

        def drain(bb, half, par):
            if half == 0:
                pltpu.make_async_copy(
                    m_h.at[idx_v.at[bb, pl.ds(0, 128)]],
                    bufs[par].at[pl.ds(0, 128)], gsems[par]).wait()
            else:
                pltpu.make_async_copy(
                    m_h.at[idx_v.at[bb, pl.ds(128, LS - 128)]],
                    bufs[par].at[pl.ds(0, LS - 128)], gsems[par]).wait()

        def accum(par, nrows, acc):
            buf = bufs[par]
            for j in range(nrows):
                off = 0 if j % 2 == 0 else 64
                for q in range(4):
                    acc[q] = acc[q] + buf[j, pl.ds(off + 16 * q, 16)]
            return acc

        def wb(bb, h, sem):
            pltpu.async_copy(h, out_h.at[pl.ds(base + bb, 1)], sem)

        def wb_wait(bb, h, sem):
            pltpu.make_async_copy(
                h, out_h.at[pl.ds(base + bb, 1)], sem).wait()

        fire(0, 0, 0)
        fire(0, 1, 1)
        fire(1, 0, 2)

        def body(i, carry):
            b0 = 2 * i

            def fire_ahead(par):
                half = (par + 3) % 2
                bn = b0 + (par + 3) // 2

                @pl.when(bn < BPW)
                def _():
                    fire(bn, half, (par + 3) % 4)

            drain(b0, 0, 0)
            acc = accum(0, 128, [jnp.zeros((16,), jnp.float32)
                                 for _ in range(4)])
            fire_ahead(0)
            drain(b0, 1, 1)
            acc = accum(1, LS - 128, acc)
            fire_ahead(1)

            @pl.when(i > 0)
            def _():
                wb_wait(b0 - 2, hA, wsemA)

            for q in range(4):
                hA[0, pl.ds(16 * q, 16)] = acc[q]
            wb(b0, hA, wsemA)

            drain(b0 + 1, 0, 2)
            acc = accum(2, 128, [jnp.zeros((16,), jnp.float32)
                                 for _ in range(4)])
            fire_ahead(2)
            drain(b0 + 1, 1, 3)
            acc = accum(3, LS - 128, acc)
            fire_ahead(3)

            @pl.when(i > 0)
            def _():
                wb_wait(b0 - 1, hB, wsemB)

            for q in range(4):
                hB[0, pl.ds(16 * q, 16)] = acc[q]
            wb(b0 + 1, hB, wsemB)
            return carry

        lax.fori_loop(0, BPW // 2, body, 0)
        wb_wait(BPW - 2, hA, wsemA)
        wb_wait(BPW - 1, hB, wsemB)

    return k(idxb, mtab)


def _partial_mm(x3, w3):
    def body(x_r, w_r, out_r, acc_r):
        kk = pl.program_id(1)

        @pl.when(kk == 0)
        def _():
            acc_r[...] = jnp.zeros_like(acc_r)

        a = acc_r[...]
        for p in range(0, TT, 2):
            x2 = jnp.concatenate([x_r[p], x_r[p + 1]], axis=-1)
            w2 = jnp.concatenate([w_r[p], w_r[p + 1]], axis=0)
            a = a + jnp.dot(x2, w2, preferred_element_type=jnp.float32)
        acc_r[...] = a

        @pl.when(kk == NT - 1)
        def _():
            out_r[...] = acc_r[...]

    return pl.pallas_call(
        body,
        grid=(B // BM, NT),
        in_specs=[
            pl.BlockSpec((TT, BM, D), lambda i, k: (k, i, 0)),
            pl.BlockSpec((TT, D, 64), lambda i, k: (k, 0, 0)),
        ],
        out_specs=pl.BlockSpec((BM, 64), lambda i, k: (i, 0)),
        out_shape=jax.ShapeDtypeStruct((B, 64), jnp.float32),
        scratch_shapes=[pltpu.VMEM((BM, 64), jnp.float32)],
        compiler_params=pltpu.CompilerParams(
            dimension_semantics=("parallel", "arbitrary")),
    )(x3, w3)


def _tail(p0, p1, p2, b1, W2, b2, W3, b3):
    def body(p0_r, p1_r, p2_r, b1_r, w2_r, b2_r, w3_r, b3_r, out_r):
        h = p0_r[...] + p1_r[...] + p2_r[...]
        h = jnp.maximum(h + b1_r[...], 0.0)
        h = jnp.maximum(
            jnp.dot(h, w2_r[...], preferred_element_type=jnp.float32)
            + b2_r[...], 0.0)
        z = jnp.dot(h, w3_r[...], preferred_element_type=jnp.float32) + b3_r[...]
        out_r[...] = jax.nn.sigmoid(z)

    return pl.pallas_call(
        body,
        grid=(B // 1024,),
        in_specs=[
            pl.BlockSpec((1024, 64), lambda i: (i, 0)),
            pl.BlockSpec((1024, 64), lambda i: (i, 0)),
            pl.BlockSpec((1024, 64), lambda i: (i, 0)),
            pl.BlockSpec((1, 64), lambda i: (0, 0)),
            pl.BlockSpec((64, 32), lambda i: (0, 0)),
            pl.BlockSpec((1, 32), lambda i: (0, 0)),
            pl.BlockSpec((32, 1), lambda i: (0, 0)),
            pl.BlockSpec((1, 1), lambda i: (0, 0)),
        ],
        out_specs=pl.BlockSpec((1024, 1), lambda i: (i, 0)),
        out_shape=jax.ShapeDtypeStruct((B, 1), jnp.float32),
    )(p0, p1, p2, b1, W2, b2, W3, b3)


def kernel(gene_input, smiles_input, gene_table, smiles_table,
           W1, b1, W2, b2, W3, b3):
    g_t = gene_input.T
    W1g = W1[:LG * D].reshape(LG, D, 64)
    W1s3 = W1[LG * D:].reshape(LS, D, 64)

    m2 = _m_prep(smiles_table, W1s3)
    midx = (jnp.arange(LS, dtype=jnp.int32)[None, :] // 2) * SV + smiles_input
    midxb = midx.reshape(NW, BPW, LS)
    hs = _sc_msum(midxb, m2.reshape(LS // 2 * SV, D))

    partials = [hs]
    for half in range(2):
        idx = g_t[half * TC_:(half + 1) * TC_].reshape(NW, NCH, CH)
        g = _sc_gather(idx, gene_table)
        partials.append(_partial_mm(g.reshape(TC_, B, D), W1g[half * TC_:(half + 1) * TC_]))

    return _tail(partials[1], partials[2], partials[0],
                 b1.reshape(1, 64), W2, b2.reshape(1, 32), W3,
                 b3.reshape(1, 1))


# speedup vs baseline: 1.0082x; 1.0082x over previous
import functools

import jax
import jax.numpy as jnp
from jax import lax
from jax.experimental import pallas as pl
from jax.experimental.pallas import tpu as pltpu
from jax.experimental.pallas import tpu_sc as plsc

B = 4096
LG = 200
LS = 200
D = 128
SV = 1000
TC_ = 100
NW = 32
CH = 128
NBUF = 4
N_LOOK = B * TC_
PER_W = N_LOOK // NW
NCH = PER_W // CH

BM = 256
TT = 20
NT = TC_ // TT

BPW = B // NW


def _sc_gather(idx, tab):
    mesh = plsc.VectorSubcoreMesh(core_axis_name="c", subcore_axis_name="s")

    @functools.partial(
        pl.kernel,
        out_type=jax.ShapeDtypeStruct((N_LOOK, D), jnp.float32),
        mesh=mesh,
        scratch_types=[
            pltpu.VMEM((NCH, CH), jnp.int32),
        ] + [pltpu.VMEM((CH, D), jnp.float32)] * NBUF
          + [pltpu.SemaphoreType.DMA] * (2 * NBUF),
    )
    def k(idx_h, tab_h, out_h, idx_v, *bufsem):
        bufs = bufsem[:NBUF]
        gsems = bufsem[NBUF:2 * NBUF]
        wsems = bufsem[2 * NBUF:]
        wid = lax.axis_index("s") * 2 + lax.axis_index("c")
        base = wid * PER_W

        pltpu.sync_copy(idx_h.at[wid], idx_v)

        def fire(s, b):
            pltpu.async_copy(tab_h.at[idx_v.at[s]], bufs[b], gsems[b])

        def drain(s, b):
            pltpu.make_async_copy(
                tab_h.at[idx_v.at[s]], bufs[b], gsems[b]).wait()

        def wb(s, b):
            pltpu.async_copy(
                bufs[b], out_h.at[pl.ds(base + s * CH, CH)], wsems[b])

        def wb_wait(s, b):
            pltpu.make_async_copy(
                bufs[b], out_h.at[pl.ds(base + s * CH, CH)], wsems[b]).wait()

        for j in range(NBUF - 1):
            fire(j, j)
        for s in range(NCH):
            b = s % NBUF
            drain(s, b)
            wb(s, b)
            nxt = s + NBUF - 1
            if nxt < NCH:
                nb = nxt % NBUF
                if s >= 1:
                    wb_wait(s - 1, nb)
                fire(nxt, nb)
        for s in range(NCH - NBUF, NCH):
            wb_wait(s, s % NBUF)

    return k(idx, tab)


def _m_prep(stab, W1s3):
    def body(x_r, w_r, out_r):
        w = jnp.concatenate([w_r[0], w_r[1]], axis=-1)
        out_r[0] = jnp.dot(x_r[...], w, preferred_element_type=jnp.float32)

    return pl.pallas_call(
        body,
        grid=(LS // 2,),
        in_specs=[
            pl.BlockSpec((SV, D), lambda g: (0, 0)),
            pl.BlockSpec((2, D, 64), lambda g: (g, 0, 0)),
        ],
        out_specs=pl.BlockSpec((1, SV, D), lambda g: (g, 0, 0)),
        out_shape=jax.ShapeDtypeStruct((LS // 2, SV, D), jnp.float32),
    )(stab, W1s3)


def _sc_msum(idxb, mtab):
    mesh = plsc.VectorSubcoreMesh(core_axis_name="c", subcore_axis_name="s")

    @functools.partial(
        pl.kernel,
        out_type=jax.ShapeDtypeStruct((B, 64), jnp.float32),
        mesh=mesh,
        scratch_types=[
            pltpu.VMEM((BPW, LS), jnp.int32),
            pltpu.VMEM((LS, D), jnp.float32),
            pltpu.VMEM((LS, D), jnp.float32),
            pltpu.VMEM((BPW, 64), jnp.float32),
            pltpu.SemaphoreType.DMA,
            pltpu.SemaphoreType.DMA,
            pltpu.SemaphoreType.DMA,
        ],
    )
    def k(idx_h, m_h, out_h, idx_v, buf0, buf1, hst,
          gsem0, gsem1, wsem):
        wid = lax.axis_index("s") * 2 + lax.axis_index("c")
        base = wid * BPW

        pltpu.sync_copy(idx_h.at[wid], idx_v)

        bufs = (buf0, buf1)
        gsems = (gsem0, gsem1)

        def fire(bb, par):
            pltpu.async_copy(m_h.at[idx_v.at[bb, pl.ds(0, 128)]],
                             bufs[par].at[pl.ds(0, 128)], gsems[par])
            pltpu.async_copy(m_h.at[idx_v.at[bb, pl.ds(128, LS - 128)]],
                             bufs[par].at[pl.ds(128, LS - 128)], gsems[par])

        def drain(bb, par):
            pltpu.make_async_copy(
                m_h.at[idx_v.at[bb, pl.ds(0, 128)]],
                bufs[par].at[pl.ds(0, 128)], gsems[par]).wait()
            pltpu.make_async_copy(
                m_h.at[idx_v.at[bb, pl.ds(128, LS - 128)]],
                bufs[par].at[pl.ds(128, LS - 128)], gsems[par]).wait()

        def proc(bb, par):
            drain(bb, par)
            buf = bufs[par]
            acc = [jnp.zeros((16,), jnp.float32) for _ in range(4)]
            for j in range(LS):
                off = 0 if j % 2 == 0 else 64
                for q in range(4):
                    acc[q] = acc[q] + buf[j, pl.ds(off + 16 * q, 16)]

            @pl.when(bb + 2 < BPW)
            def _():
                fire(bb + 2, par)

            for q in range(4):
                hst[bb, pl.ds(16 * q, 16)] = acc[q]

        fire(0, 0)
        fire(1, 1)

        def body(i, carry):
            b0 = 2 * i
            proc(b0, 0)
            proc(b0 + 1, 1)
            return carry

        lax.fori_loop(0, BPW // 2, body, 0)
        pltpu.async_copy(hst, out_h.at[pl.ds(base, BPW)], wsem)
        pltpu.make_async_copy(hst, out_h.at[pl.ds(base, BPW)], wsem).wait()

    return k(idxb, mtab)


def _partial_mm(x3, w3):
    def body(x_r, w_r, out_r, acc_r):
        kk = pl.program_id(1)

        @pl.when(kk == 0)
        def _():
            acc_r[...] = jnp.zeros_like(acc_r)

        a = acc_r[...]
        for p in range(0, TT, 2):
            x2 = jnp.concatenate([x_r[p], x_r[p + 1]], axis=-1)
            w2 = jnp.concatenate([w_r[p], w_r[p + 1]], axis=0)
            a = a + jnp.dot(x2, w2, preferred_element_type=jnp.float32)
        acc_r[...] = a

        @pl.when(kk == NT - 1)
        def _():
            out_r[...] = acc_r[...]

    return pl.pallas_call(
        body,
        grid=(B // BM, NT),
        in_specs=[
            pl.BlockSpec((TT, BM, D), lambda i, k: (k, i, 0)),
            pl.BlockSpec((TT, D, 64), lambda i, k: (k, 0, 0)),
        ],
        out_specs=pl.BlockSpec((BM, 64), lambda i, k: (i, 0)),
        out_shape=jax.ShapeDtypeStruct((B, 64), jnp.float32),
        scratch_shapes=[pltpu.VMEM((BM, 64), jnp.float32)],
        compiler_params=pltpu.CompilerParams(
            dimension_semantics=("parallel", "arbitrary")),
    )(x3, w3)


def _tail(p0, p1, p2, b1, W2, b2, W3, b3):
    def body(p0_r, p1_r, p2_r, b1_r, w2_r, b2_r, w3_r, b3_r, out_r):
        h = p0_r[...] + p1_r[...] + p2_r[...]
        h = jnp.maximum(h + b1_r[...], 0.0)
        h = jnp.maximum(
            jnp.dot(h, w2_r[...], preferred_element_type=jnp.float32)
            + b2_r[...], 0.0)
        z = jnp.dot(h, w3_r[...], preferred_element_type=jnp.float32) + b3_r[...]
        out_r[...] = jax.nn.sigmoid(z)

    return pl.pallas_call(
        body,
        grid=(B // 1024,),
        in_specs=[
            pl.BlockSpec((1024, 64), lambda i: (i, 0)),
            pl.BlockSpec((1024, 64), lambda i: (i, 0)),
            pl.BlockSpec((1024, 64), lambda i: (i, 0)),
            pl.BlockSpec((1, 64), lambda i: (0, 0)),
            pl.BlockSpec((64, 32), lambda i: (0, 0)),
            pl.BlockSpec((1, 32), lambda i: (0, 0)),
            pl.BlockSpec((32, 1), lambda i: (0, 0)),
            pl.BlockSpec((1, 1), lambda i: (0, 0)),
        ],
        out_specs=pl.BlockSpec((1024, 1), lambda i: (i, 0)),
        out_shape=jax.ShapeDtypeStruct((B, 1), jnp.float32),
    )(p0, p1, p2, b1, W2, b2, W3, b3)


def kernel(gene_input, smiles_input, gene_table, smiles_table,
           W1, b1, W2, b2, W3, b3):
    g_t = gene_input.T
    W1g = W1[:LG * D].reshape(LG, D, 64)
    W1s3 = W1[LG * D:].reshape(LS, D, 64)

    m2 = _m_prep(smiles_table, W1s3)
    midx = (jnp.arange(LS, dtype=jnp.int32)[None, :] // 2) * SV + smiles_input
    midxb = midx.reshape(NW, BPW, LS)
    hs = _sc_msum(midxb, m2.reshape(LS // 2 * SV, D))

    partials = [hs]
    for half in range(2):
        idx = g_t[half * TC_:(half + 1) * TC_].reshape(NW, NCH, CH)
        g = _sc_gather(idx, gene_table)
        partials.append(_partial_mm(g.reshape(TC_, B, D), W1g[half * TC_:(half + 1) * TC_]))

    return _tail(partials[1], partials[2], partials[0],
                 b1.reshape(1, 64), W2, b2.reshape(1, 32), W3,
                 b3.reshape(1, 1))


# speedup vs baseline: 1.0302x; 1.0218x over previous
import functools

import jax
import jax.numpy as jnp
from jax import lax
from jax.experimental import pallas as pl
from jax.experimental.pallas import tpu as pltpu
from jax.experimental.pallas import tpu_sc as plsc

B = 4096
LG = 200
LS = 200
D = 128
SV = 1000
TC_ = 100
NW = 32
CH = 128
NBUF = 4
N_LOOK = B * TC_
PER_W = N_LOOK // NW
NCH = PER_W // CH

BM = 256
TT = 20
NT = TC_ // TT

BPW = B // NW


def _sc_gather(idx, tab):
    mesh = plsc.VectorSubcoreMesh(core_axis_name="c", subcore_axis_name="s")

    @functools.partial(
        pl.kernel,
        out_type=jax.ShapeDtypeStruct((N_LOOK, D), jnp.float32),
        mesh=mesh,
        scratch_types=[
            pltpu.VMEM((NCH, CH), jnp.int32),
        ] + [pltpu.VMEM((CH, D), jnp.float32)] * NBUF
          + [pltpu.SemaphoreType.DMA] * (2 * NBUF),
    )
    def k(idx_h, tab_h, out_h, idx_v, *bufsem):
        bufs = bufsem[:NBUF]
        gsems = bufsem[NBUF:2 * NBUF]
        wsems = bufsem[2 * NBUF:]
        wid = lax.axis_index("s") * 2 + lax.axis_index("c")
        base = wid * PER_W

        pltpu.sync_copy(idx_h.at[wid], idx_v)

        def fire(s, b):
            pltpu.async_copy(tab_h.at[idx_v.at[s]], bufs[b], gsems[b])

        def drain(s, b):
            pltpu.make_async_copy(
                tab_h.at[idx_v.at[s]], bufs[b], gsems[b]).wait()

        def wb(s, b):
            pltpu.async_copy(
                bufs[b], out_h.at[pl.ds(base + s * CH, CH)], wsems[b])

        def wb_wait(s, b):
            pltpu.make_async_copy(
                bufs[b], out_h.at[pl.ds(base + s * CH, CH)], wsems[b]).wait()

        for j in range(NBUF - 1):
            fire(j, j)
        for s in range(NCH):
            b = s % NBUF
            drain(s, b)
            wb(s, b)
            nxt = s + NBUF - 1
            if nxt < NCH:
                nb = nxt % NBUF
                if s >= 1:
                    wb_wait(s - 1, nb)
                fire(nxt, nb)
        for s in range(NCH - NBUF, NCH):
            wb_wait(s, s % NBUF)

    return k(idx, tab)


def _m_prep(stab, W1s3):
    def body(x_r, w_r, out_r):
        w = jnp.concatenate([w_r[0], w_r[1]], axis=-1)
        out_r[0] = jnp.dot(x_r[...], w, preferred_element_type=jnp.float32)

    return pl.pallas_call(
        body,
        grid=(LS // 2,),
        in_specs=[
            pl.BlockSpec((SV, D), lambda g: (0, 0)),
            pl.BlockSpec((2, D, 64), lambda g: (g, 0, 0)),
        ],
        out_specs=pl.BlockSpec((1, SV, D), lambda g: (g, 0, 0)),
        out_shape=jax.ShapeDtypeStruct((LS // 2, SV, D), jnp.float32),
    )(stab, W1s3)


def _sc_msum(idxb, mtab):
    mesh = plsc.VectorSubcoreMesh(core_axis_name="c", subcore_axis_name="s")

    @functools.partial(
        pl.kernel,
        out_type=jax.ShapeDtypeStruct((B, 64), jnp.float32),
        mesh=mesh,
        scratch_types=[
            pltpu.VMEM((BPW, LS), jnp.int32),
            pltpu.VMEM((LS, D), jnp.float32),
            pltpu.VMEM((LS, D), jnp.float32),
            pltpu.VMEM((1, 64), jnp.float32),
            pltpu.VMEM((1, 64), jnp.float32),
            pltpu.SemaphoreType.DMA,
            pltpu.SemaphoreType.DMA,
            pltpu.SemaphoreType.DMA,
            pltpu.SemaphoreType.DMA,
        ],
    )
    def k(idx_h, m_h, out_h, idx_v, buf0, buf1, h0, h1,
          gsem0, gsem1, wsem0, wsem1):
        wid = lax.axis_index("s") * 2 + lax.axis_index("c")
        base = wid * BPW

        pltpu.sync_copy(idx_h.at[wid], idx_v)

        bufs = (buf0, buf1)
        hs = (h0, h1)
        gsems = (gsem0, gsem1)
        wsems = (wsem0, wsem1)

        def fire(bb, par):
            pltpu.async_copy(m_h.at[idx_v.at[bb, pl.ds(0, 128)]],
                             bufs[par].at[pl.ds(0, 128)], gsems[par])
            pltpu.async_copy(m_h.at[idx_v.at[bb, pl.ds(128, LS - 128)]],
                             bufs[par].at[pl.ds(128, LS - 128)], gsems[par])

        def drain(bb, par):
            pltpu.make_async_copy(
                m_h.at[idx_v.at[bb, pl.ds(0, 128)]],
                bufs[par].at[pl.ds(0, 128)], gsems[par]).wait()
            pltpu.make_async_copy(
                m_h.at[idx_v.at[bb, pl.ds(128, LS - 128)]],
                bufs[par].at[pl.ds(128, LS - 128)], gsems[par]).wait()

        def wb(bb, par):
            pltpu.async_copy(hs[par], out_h.at[pl.ds(base + bb, 1)],
                             wsems[par])

        def wb_wait(bb, par):
            pltpu.make_async_copy(
                hs[par], out_h.at[pl.ds(base + bb, 1)], wsems[par]).wait()

        def process(bb, par, first):
            buf = bufs[par]
            drain(bb, par)

            acc = [jnp.zeros((16,), jnp.float32) for _ in range(4)]
            for j in range(LS):
                off = 0 if j % 2 == 0 else 64
                for q in range(4):
                    acc[q] = acc[q] + buf[j, pl.ds(off + 16 * q, 16)]

            @pl.when(bb + 2 < BPW)
            def _():
                fire(bb + 2, par)

            @pl.when(jnp.logical_not(first))
            def _():
                wb_wait(bb - 2, par)

            for q in range(4):
                hs[par][0, pl.ds(16 * q, 16)] = acc[q]
            wb(bb, par)

        fire(0, 0)
        fire(1, 1)

        def body(i, carry):
            b0 = 2 * i
            process(b0, 0, i == 0)
            process(b0 + 1, 1, i == 0)
            return carry

        lax.fori_loop(0, BPW // 2, body, 0)
        wb_wait(BPW - 2, 0)
        wb_wait(BPW - 1, 1)

    return k(idxb, mtab)


def _partial_mm(x3, w3):
    def body(x_r, w_r, out_r, acc_r):
        kk = pl.program_id(1)

        @pl.when(kk == 0)
        def _():
            acc_r[...] = jnp.zeros_like(acc_r)

        a = acc_r[...]
        for p in range(0, TT, 2):
            x2 = jnp.concatenate([x_r[p], x_r[p + 1]], axis=-1)
            w2 = jnp.concatenate([w_r[p], w_r[p + 1]], axis=0)
            a = a + jnp.dot(x2, w2, preferred_element_type=jnp.float32)
        acc_r[...] = a

        @pl.when(kk == NT - 1)
        def _():
            out_r[...] = acc_r[...]

    return pl.pallas_call(
        body,
        grid=(B // BM, NT),
        in_specs=[
            pl.BlockSpec((TT, BM, D), lambda i, k: (k, i, 0)),
            pl.BlockSpec((TT, D, 64), lambda i, k: (k, 0, 0)),
        ],
        out_specs=pl.BlockSpec((BM, 64), lambda i, k: (i, 0)),
        out_shape=jax.ShapeDtypeStruct((B, 64), jnp.float32),
        scratch_shapes=[pltpu.VMEM((BM, 64), jnp.float32)],
        compiler_params=pltpu.CompilerParams(
            dimension_semantics=("parallel", "arbitrary")),
    )(x3, w3)


def _tail(p0, p1, p2, b1, W2, b2, W3, b3):
    def body(p0_r, p1_r, p2_r, b1_r, w2_r, b2_r, w3_r, b3_r, out_r):
        h = p0_r[...] + p1_r[...] + p2_r[...]
        h = jnp.maximum(h + b1_r[...], 0.0)
        h = jnp.maximum(
            jnp.dot(h, w2_r[...], preferred_element_type=jnp.float32)
            + b2_r[...], 0.0)
        z = jnp.dot(h, w3_r[...], preferred_element_type=jnp.float32) + b3_r[...]
        out_r[...] = jax.nn.sigmoid(z)

    return pl.pallas_call(
        body,
        grid=(B // 1024,),
        in_specs=[
            pl.BlockSpec((1024, 64), lambda i: (i, 0)),
            pl.BlockSpec((1024, 64), lambda i: (i, 0)),
            pl.BlockSpec((1024, 64), lambda i: (i, 0)),
            pl.BlockSpec((1, 64), lambda i: (0, 0)),
            pl.BlockSpec((64, 32), lambda i: (0, 0)),
            pl.BlockSpec((1, 32), lambda i: (0, 0)),
            pl.BlockSpec((32, 1), lambda i: (0, 0)),
            pl.BlockSpec((1, 1), lambda i: (0, 0)),
        ],
        out_specs=pl.BlockSpec((1024, 1), lambda i: (i, 0)),
        out_shape=jax.ShapeDtypeStruct((B, 1), jnp.float32),
    )(p0, p1, p2, b1, W2, b2, W3, b3)


def kernel(gene_input, smiles_input, gene_table, smiles_table,
           W1, b1, W2, b2, W3, b3):
    g_t = gene_input.T
    W1g = W1[:LG * D].reshape(LG, D, 64)
    W1s3 = W1[LG * D:].reshape(LS, D, 64)

    m2 = _m_prep(smiles_table, W1s3)
    midx = (jnp.arange(LS, dtype=jnp.int32)[None, :] // 2) * SV + smiles_input
    midxb = midx.reshape(NW, BPW, LS)
    hs = _sc_msum(midxb, m2.reshape(LS // 2 * SV, D))

    partials = [hs]
    for half in range(2):
        idx = g_t[half * TC_:(half + 1) * TC_].reshape(NW, NCH, CH)
        g = _sc_gather(idx, gene_table)
        partials.append(_partial_mm(g.reshape(TC_, B, D), W1g[half * TC_:(half + 1) * TC_]))

    return _tail(partials[1], partials[2], partials[0],
                 b1.reshape(1, 64), W2, b2.reshape(1, 32), W3,
                 b3.reshape(1, 1))


# speedup vs baseline: 1.1134x; 1.0807x over previous
import functools

import jax
import jax.numpy as jnp
from jax import lax
from jax.experimental import pallas as pl
from jax.experimental.pallas import tpu as pltpu
from jax.experimental.pallas import tpu_sc as plsc

B = 4096
LG = 200
LS = 200
D = 128
SV = 1000
TC_ = 200
NW = 32
CH = 128
NBUF = 4
N_LOOK = B * TC_
PER_W = N_LOOK // NW
NCH = PER_W // CH

BM = 256
TT = 20
NT = TC_ // TT

BPW = B // NW


def _sc_gather(idx, tab):
    mesh = plsc.VectorSubcoreMesh(core_axis_name="c", subcore_axis_name="s")

    @functools.partial(
        pl.kernel,
        out_type=jax.ShapeDtypeStruct((N_LOOK, D), jnp.float32),
        mesh=mesh,
        scratch_types=[
            pltpu.VMEM((NCH, CH), jnp.int32),
        ] + [pltpu.VMEM((CH, D), jnp.float32)] * NBUF
          + [pltpu.SemaphoreType.DMA] * (2 * NBUF),
    )
    def k(idx_h, tab_h, out_h, idx_v, *bufsem):
        bufs = bufsem[:NBUF]
        gsems = bufsem[NBUF:2 * NBUF]
        wsems = bufsem[2 * NBUF:]
        wid = lax.axis_index("s") * 2 + lax.axis_index("c")
        base = wid * PER_W

        pltpu.sync_copy(idx_h.at[wid], idx_v)

        def fire(s, b):
            pltpu.async_copy(tab_h.at[idx_v.at[s]], bufs[b], gsems[b])

        def drain(s, b):
            pltpu.make_async_copy(
                tab_h.at[idx_v.at[s]], bufs[b], gsems[b]).wait()

        def wb(s, b):
            pltpu.async_copy(
                bufs[b], out_h.at[pl.ds(base + s * CH, CH)], wsems[b])

        def wb_wait(s, b):
            pltpu.make_async_copy(
                bufs[b], out_h.at[pl.ds(base + s * CH, CH)], wsems[b]).wait()

        for j in range(NBUF - 1):
            fire(j, j)
        for s in range(NCH):
            b = s % NBUF
            drain(s, b)
            wb(s, b)
            nxt = s + NBUF - 1
            if nxt < NCH:
                nb = nxt % NBUF
                if s >= 1:
                    wb_wait(s - 1, nb)
                fire(nxt, nb)
        for s in range(NCH - NBUF, NCH):
            wb_wait(s, s % NBUF)

    return k(idx, tab)


def _m_prep(stab, W1s3):
    def body(x_r, w_r, out_r):
        w = jnp.concatenate([w_r[0], w_r[1]], axis=-1)
        out_r[0] = jnp.dot(x_r[...], w, preferred_element_type=jnp.float32)

    return pl.pallas_call(
        body,
        grid=(LS // 2,),
        in_specs=[
            pl.BlockSpec((SV, D), lambda g: (0, 0)),
            pl.BlockSpec((2, D, 64), lambda g: (g, 0, 0)),
        ],
        out_specs=pl.BlockSpec((1, SV, D), lambda g: (g, 0, 0)),
        out_shape=jax.ShapeDtypeStruct((LS // 2, SV, D), jnp.float32),
    )(stab, W1s3)


def _sc_msum(idxb, mtab):
    mesh = plsc.VectorSubcoreMesh(core_axis_name="c", subcore_axis_name="s")

    @functools.partial(
        pl.kernel,
        out_type=jax.ShapeDtypeStruct((B, 64), jnp.float32),
        mesh=mesh,
        scratch_types=[
            pltpu.VMEM((BPW, LS), jnp.int32),
            pltpu.VMEM((LS, D), jnp.float32),
            pltpu.VMEM((LS, D), jnp.float32),
            pltpu.VMEM((1, 64), jnp.float32),
            pltpu.VMEM((1, 64), jnp.float32),
            pltpu.SemaphoreType.DMA,
            pltpu.SemaphoreType.DMA,
            pltpu.SemaphoreType.DMA,
            pltpu.SemaphoreType.DMA,
        ],
    )
    def k(idx_h, m_h, out_h, idx_v, buf0, buf1, h0, h1,
          gsem0, gsem1, wsem0, wsem1):
        wid = lax.axis_index("s") * 2 + lax.axis_index("c")
        base = wid * BPW

        pltpu.sync_copy(idx_h.at[wid], idx_v)

        bufs = (buf0, buf1)
        hs = (h0, h1)
        gsems = (gsem0, gsem1)
        wsems = (wsem0, wsem1)

        def fire(bb, par):
            pltpu.async_copy(m_h.at[idx_v.at[bb, pl.ds(0, 128)]],
                             bufs[par].at[pl.ds(0, 128)], gsems[par])
            pltpu.async_copy(m_h.at[idx_v.at[bb, pl.ds(128, LS - 128)]],
                             bufs[par].at[pl.ds(128, LS - 128)], gsems[par])

        def drain(bb, par):
            pltpu.make_async_copy(
                m_h.at[idx_v.at[bb, pl.ds(0, 128)]],
                bufs[par].at[pl.ds(0, 128)], gsems[par]).wait()
            pltpu.make_async_copy(
                m_h.at[idx_v.at[bb, pl.ds(128, LS - 128)]],
                bufs[par].at[pl.ds(128, LS - 128)], gsems[par]).wait()

        def wb(bb, par):
            pltpu.async_copy(hs[par], out_h.at[pl.ds(base + bb, 1)],
                             wsems[par])

        def wb_wait(bb, par):
            pltpu.make_async_copy(
                hs[par], out_h.at[pl.ds(base + bb, 1)], wsems[par]).wait()

        def process(bb, par, first):
            buf = bufs[par]
            drain(bb, par)

            acc = [jnp.zeros((16,), jnp.float32) for _ in range(4)]
            for j in range(LS):
                off = 0 if j % 2 == 0 else 64
                for q in range(4):
                    acc[q] = acc[q] + buf[j, pl.ds(off + 16 * q, 16)]

            @pl.when(bb + 2 < BPW)
            def _():
                fire(bb + 2, par)

            @pl.when(jnp.logical_not(first))
            def _():
                wb_wait(bb - 2, par)

            for q in range(4):
                hs[par][0, pl.ds(16 * q, 16)] = acc[q]
            wb(bb, par)

        fire(0, 0)
        fire(1, 1)

        def body(i, carry):
            b0 = 2 * i
            process(b0, 0, i == 0)
            process(b0 + 1, 1, i == 0)
            return carry

        lax.fori_loop(0, BPW // 2, body, 0)
        wb_wait(BPW - 2, 0)
        wb_wait(BPW - 1, 1)

    return k(idxb, mtab)


def _partial_mm(x3, w3):
    def body(x_r, w_r, out_r, acc_r):
        kk = pl.program_id(1)

        @pl.when(kk == 0)
        def _():
            acc_r[...] = jnp.zeros_like(acc_r)

        a = acc_r[...]
        for p in range(0, TT, 2):
            x2 = jnp.concatenate([x_r[p], x_r[p + 1]], axis=-1)
            w2 = jnp.concatenate([w_r[p], w_r[p + 1]], axis=0)
            a = a + jnp.dot(x2, w2, preferred_element_type=jnp.float32)
        acc_r[...] = a

        @pl.when(kk == NT - 1)
        def _():
            out_r[...] = acc_r[...]

    return pl.pallas_call(
        body,
        grid=(B // BM, NT),
        in_specs=[
            pl.BlockSpec((TT, BM, D), lambda i, k: (k, i, 0)),
            pl.BlockSpec((TT, D, 64), lambda i, k: (k, 0, 0)),
        ],
        out_specs=pl.BlockSpec((BM, 64), lambda i, k: (i, 0)),
        out_shape=jax.ShapeDtypeStruct((B, 64), jnp.float32),
        scratch_shapes=[pltpu.VMEM((BM, 64), jnp.float32)],
        compiler_params=pltpu.CompilerParams(
            dimension_semantics=("parallel", "arbitrary")),
    )(x3, w3)


def _tail(p0, p1, b1, W2, b2, W3, b3):
    def body(p0_r, p1_r, b1_r, w2_r, b2_r, w3_r, b3_r, out_r):
        h = p0_r[...] + p1_r[...]
        h = jnp.maximum(h + b1_r[...], 0.0)
        h = jnp.maximum(
            jnp.dot(h, w2_r[...], preferred_element_type=jnp.float32)
            + b2_r[...], 0.0)
        z = jnp.dot(h, w3_r[...], preferred_element_type=jnp.float32) + b3_r[...]
        out_r[...] = jax.nn.sigmoid(z)

    return pl.pallas_call(
        body,
        grid=(B // 1024,),
        in_specs=[
            pl.BlockSpec((1024, 64), lambda i: (i, 0)),
            pl.BlockSpec((1024, 64), lambda i: (i, 0)),
            pl.BlockSpec((1, 64), lambda i: (0, 0)),
            pl.BlockSpec((64, 32), lambda i: (0, 0)),
            pl.BlockSpec((1, 32), lambda i: (0, 0)),
            pl.BlockSpec((32, 1), lambda i: (0, 0)),
            pl.BlockSpec((1, 1), lambda i: (0, 0)),
        ],
        out_specs=pl.BlockSpec((1024, 1), lambda i: (i, 0)),
        out_shape=jax.ShapeDtypeStruct((B, 1), jnp.float32),
    )(p0, p1, b1, W2, b2, W3, b3)


def kernel(gene_input, smiles_input, gene_table, smiles_table,
           W1, b1, W2, b2, W3, b3):
    g_t = gene_input.T
    W1g = W1[:LG * D].reshape(LG, D, 64)
    W1s3 = W1[LG * D:].reshape(LS, D, 64)

    m2 = _m_prep(smiles_table, W1s3)
    midx = (jnp.arange(LS, dtype=jnp.int32)[None, :] // 2) * SV + smiles_input
    midxb = midx.reshape(NW, BPW, LS)
    hs = _sc_msum(midxb, m2.reshape(LS // 2 * SV, D))

    idx = g_t.reshape(NW, NCH, CH)
    g = _sc_gather(idx, gene_table)
    pg = _partial_mm(g.reshape(TC_, B, D), W1g)

    return _tail(pg, hs, b1.reshape(1, 64), W2, b2.reshape(1, 32), W3,
                 b3.reshape(1, 1))
